# Initial kernel scaffold; baseline (speedup 1.0000x reference)
import jax
import jax.numpy as jnp
from jax.experimental import pallas as pl


def kernel(embeddings, position_embeddings, indices, positions):
    raise NotImplementedError("write your pallas kernel here")


# speedup vs baseline: 1.0511x; 1.0511x over previous
import functools

import jax
import jax.numpy as jnp
from jax import lax
from jax.experimental import pallas as pl
from jax.experimental.pallas import tpu as pltpu
from jax.experimental.pallas import tpu_sc as plsc

NUM_KMERS = 4096
DIM = 10000
MAX_POSITIONS = 1000
BATCH = 1024
SEQ = 20
N = BATCH * SEQ

NUM_CORES = 2
NUM_SUBCORES = 16
NUM_WORKERS = NUM_CORES * NUM_SUBCORES
ROWS_PER_WORKER = N // NUM_WORKERS
CHUNK = 4
NUM_CHUNKS = ROWS_PER_WORKER // CHUNK
LANES = 16
VECS_PER_ROW = DIM // LANES


def _sigmoid_body(pos_ref, out_ref):
    out_ref[...] = jax.nn.sigmoid(pos_ref[...])


def _make_gate_table(position_embeddings):
    return pl.pallas_call(
        _sigmoid_body,
        grid=(5,),
        in_specs=[pl.BlockSpec((200, DIM), lambda i: (i, 0))],
        out_specs=pl.BlockSpec((200, DIM), lambda i: (i, 0)),
        out_shape=jax.ShapeDtypeStruct((MAX_POSITIONS, DIM), jnp.float32),
    )(position_embeddings)


_sc_mesh = plsc.VectorSubcoreMesh(core_axis_name="c", subcore_axis_name="s")


@functools.partial(
    pl.kernel,
    mesh=_sc_mesh,
    compiler_params=pltpu.CompilerParams(use_tc_tiling_on_sc=False),
    out_type=jax.ShapeDtypeStruct((N, DIM), jnp.float32),
    scratch_types=[
        pltpu.VMEM((NUM_CHUNKS, CHUNK), jnp.int32),
        pltpu.VMEM((NUM_CHUNKS, CHUNK), jnp.int32),
        pltpu.VMEM((CHUNK, DIM), jnp.float32),
        pltpu.VMEM((CHUNK, DIM), jnp.float32),
        pltpu.SemaphoreType.DMA,
    ],
)
def _sc_gather_mul(emb_hbm, gate_hbm, idx_hbm, pos_hbm, out_hbm,
                   idx_v, pos_v, emb_v, gate_v, sem):
    wid = lax.axis_index("s") * NUM_CORES + lax.axis_index("c")
    base = wid * ROWS_PER_WORKER
    pltpu.sync_copy(idx_hbm.at[wid], idx_v)
    pltpu.sync_copy(pos_hbm.at[wid], pos_v)

    def chunk_body(g, carry):
        r0 = g * CHUNK
        ce = pltpu.async_copy(emb_hbm.at[idx_v.at[g]], emb_v, sem)
        cg = pltpu.async_copy(gate_hbm.at[pos_v.at[g]], gate_v, sem)
        ce.wait()
        cg.wait()

        def vec_body(j, c):
            off = j * LANES
            for r in range(CHUNK):
                emb_v[r, pl.ds(off, LANES)] = (
                    emb_v[r, pl.ds(off, LANES)] * gate_v[r, pl.ds(off, LANES)]
                )
            return c

        lax.fori_loop(0, VECS_PER_ROW, vec_body, 0)
        pltpu.sync_copy(emb_v, out_hbm.at[pl.ds(base + r0, CHUNK)])
        return carry

    lax.fori_loop(0, NUM_CHUNKS, chunk_body, 0)


def kernel(embeddings, position_embeddings, indices, positions):
    gate_table = _make_gate_table(position_embeddings)
    idx_3d = indices.reshape(NUM_WORKERS, NUM_CHUNKS, CHUNK)
    pos_3d = positions.reshape(NUM_WORKERS, NUM_CHUNKS, CHUNK)
    out = _sc_gather_mul(embeddings, gate_table, idx_3d, pos_3d)
    return out.reshape(BATCH, SEQ, DIM)


# speedup vs baseline: 1.1219x; 1.0674x over previous
import functools

import jax
import jax.numpy as jnp
from jax import lax
from jax.experimental import pallas as pl
from jax.experimental.pallas import tpu as pltpu
from jax.experimental.pallas import tpu_sc as plsc

NUM_KMERS = 4096
DIM = 10000
DIM_PAD = 10112
MAX_POSITIONS = 1000
BATCH = 1024
SEQ = 20
N = BATCH * SEQ

NUM_CORES = 2
NUM_SUBCORES = 16
NUM_WORKERS = NUM_CORES * NUM_SUBCORES
ROWS_PER_WORKER = N // NUM_WORKERS
CHUNK = 2
NUM_CHUNKS = ROWS_PER_WORKER // CHUNK
LANES = 16
VECS_PER_ROW = DIM // LANES


def _pad_body(in_ref, out_ref):
    out_ref[:, :DIM] = in_ref[...]
    out_ref[:, DIM:] = jnp.zeros((in_ref.shape[0], DIM_PAD - DIM), jnp.float32)


def _sigmoid_pad_body(in_ref, out_ref):
    out_ref[:, :DIM] = jax.nn.sigmoid(in_ref[...])
    out_ref[:, DIM:] = jnp.zeros((in_ref.shape[0], DIM_PAD - DIM), jnp.float32)


def _pad_table(table, body, rows_per_block):
    rows = table.shape[0]
    return pl.pallas_call(
        body,
        grid=(rows // rows_per_block,),
        in_specs=[pl.BlockSpec((rows_per_block, DIM), lambda i: (i, 0))],
        out_specs=pl.BlockSpec((rows_per_block, DIM_PAD), lambda i: (i, 0)),
        out_shape=jax.ShapeDtypeStruct((rows, DIM_PAD), jnp.float32),
    )(table)


_sc_mesh = plsc.VectorSubcoreMesh(core_axis_name="c", subcore_axis_name="s")


@functools.partial(
    pl.kernel,
    mesh=_sc_mesh,
    out_type=jax.ShapeDtypeStruct((N, DIM), jnp.float32),
    scratch_types=[
        pltpu.VMEM((NUM_CHUNKS * 16,), jnp.int32),
        pltpu.VMEM((CHUNK, DIM_PAD), jnp.float32),
        pltpu.VMEM((CHUNK, DIM_PAD), jnp.float32),
        pltpu.VMEM((CHUNK, DIM), jnp.float32),
        pltpu.SemaphoreType.DMA,
    ],
)
def _sc_gather_mul(emb_hbm, gate_hbm, packed_hbm, out_hbm,
                   packed_v, emb_v, gate_v, prod_v, sem):
    wid = lax.axis_index("s") * NUM_CORES + lax.axis_index("c")
    base = wid * ROWS_PER_WORKER
    pltpu.sync_copy(packed_hbm.at[pl.ds(wid * NUM_CHUNKS * 16, NUM_CHUNKS * 16)],
                    packed_v)

    def chunk_body(g, carry):
        r0 = g * CHUNK
        ce = pltpu.async_copy(emb_hbm.at[packed_v.at[pl.ds(g * 16, CHUNK)]], emb_v, sem)
        cg = pltpu.async_copy(gate_hbm.at[packed_v.at[pl.ds(g * 16 + 8, CHUNK)]], gate_v, sem)
        ce.wait()
        cg.wait()

        def vec_body(j, c):
            off = j * LANES
            for r in range(CHUNK):
                prod_v[r, pl.ds(off, LANES)] = (
                    emb_v[r, pl.ds(off, LANES)] * gate_v[r, pl.ds(off, LANES)]
                )
            return c

        lax.fori_loop(0, VECS_PER_ROW, vec_body, 0)
        pltpu.sync_copy(prod_v, out_hbm.at[pl.ds(base + r0, CHUNK)])
        return carry

    lax.fori_loop(0, NUM_CHUNKS, chunk_body, 0)


def kernel(embeddings, position_embeddings, indices, positions):
    emb_pad = _pad_table(embeddings, _pad_body, 128)
    gate_pad = _pad_table(position_embeddings, _sigmoid_pad_body, 200)
    idx_c = indices.reshape(NUM_WORKERS, NUM_CHUNKS, CHUNK)
    pos_c = positions.reshape(NUM_WORKERS, NUM_CHUNKS, CHUNK)
    fill = jnp.zeros((NUM_WORKERS, NUM_CHUNKS, 8 - CHUNK), jnp.int32)
    packed = jnp.concatenate([idx_c, fill, pos_c, fill], axis=-1)
    out = _sc_gather_mul(emb_pad, gate_pad, packed.reshape(NUM_WORKERS * NUM_CHUNKS * 16))
    return out.reshape(BATCH, SEQ, DIM)


# speedup vs baseline: 1.2818x; 1.1426x over previous
import functools

import jax
import jax.numpy as jnp
from jax import lax
from jax.experimental import pallas as pl
from jax.experimental.pallas import tpu as pltpu
from jax.experimental.pallas import tpu_sc as plsc

NUM_KMERS = 4096
DIM = 10000
DIM_PAD = 10112
MAX_POSITIONS = 1000
BATCH = 1024
SEQ = 20
N = BATCH * SEQ

NUM_CORES = 2
NUM_SUBCORES = 16
NUM_WORKERS = NUM_CORES * NUM_SUBCORES
ROWS_PER_WORKER = N // NUM_WORKERS
CHUNK = 2
NUM_CHUNKS = ROWS_PER_WORKER // CHUNK
LANES = 16
VECS_PER_ROW = DIM // LANES


def _pad_body(in_ref, out_ref):
    out_ref[:, :DIM] = in_ref[...]
    out_ref[:, DIM:] = jnp.zeros((in_ref.shape[0], DIM_PAD - DIM), jnp.float32)


def _sigmoid_pad_body(in_ref, out_ref):
    out_ref[:, :DIM] = jax.nn.sigmoid(in_ref[...])
    out_ref[:, DIM:] = jnp.zeros((in_ref.shape[0], DIM_PAD - DIM), jnp.float32)


def _pad_table(table, body, rows_per_block):
    rows = table.shape[0]
    return pl.pallas_call(
        body,
        grid=(rows // rows_per_block,),
        in_specs=[pl.BlockSpec((rows_per_block, DIM), lambda i: (i, 0))],
        out_specs=pl.BlockSpec((rows_per_block, DIM_PAD), lambda i: (i, 0)),
        out_shape=jax.ShapeDtypeStruct((rows, DIM_PAD), jnp.float32),
    )(table)


_sc_mesh = plsc.VectorSubcoreMesh(core_axis_name="c", subcore_axis_name="s")


@functools.partial(
    pl.kernel,
    mesh=_sc_mesh,
    out_type=jax.ShapeDtypeStruct((BATCH, SEQ, DIM), jnp.float32),
    scratch_types=[
        pltpu.VMEM((NUM_CHUNKS * 16,), jnp.int32),
        pltpu.VMEM((CHUNK, DIM_PAD), jnp.float32),
        pltpu.VMEM((CHUNK, DIM_PAD), jnp.float32),
        pltpu.VMEM((CHUNK, DIM), jnp.float32),
        pltpu.SemaphoreType.DMA,
    ],
)
def _sc_gather_mul(emb_hbm, gate_hbm, packed_hbm, out_hbm,
                   packed_v, emb_v, gate_v, prod_v, sem):
    wid = lax.axis_index("s") * NUM_CORES + lax.axis_index("c")
    base = wid * ROWS_PER_WORKER
    pltpu.sync_copy(packed_hbm.at[pl.ds(wid * NUM_CHUNKS * 16, NUM_CHUNKS * 16)],
                    packed_v)

    def chunk_body(g, carry):
        r0 = g * CHUNK
        ce = pltpu.async_copy(emb_hbm.at[packed_v.at[pl.ds(g * 16, CHUNK)]], emb_v, sem)
        cg = pltpu.async_copy(gate_hbm.at[packed_v.at[pl.ds(g * 16 + 8, CHUNK)]], gate_v, sem)
        ce.wait()
        cg.wait()

        def vec_body(j, c):
            off = j * LANES
            for r in range(CHUNK):
                prod_v[r, pl.ds(off, LANES)] = (
                    emb_v[r, pl.ds(off, LANES)] * gate_v[r, pl.ds(off, LANES)]
                )
            return c

        lax.fori_loop(0, VECS_PER_ROW, vec_body, 0)
        row = base + r0
        pltpu.sync_copy(prod_v, out_hbm.at[row // SEQ, pl.ds(row % SEQ, CHUNK)])
        return carry

    lax.fori_loop(0, NUM_CHUNKS, chunk_body, 0)


def kernel(embeddings, position_embeddings, indices, positions):
    emb_pad = _pad_table(embeddings, _pad_body, 128)
    gate_pad = _pad_table(position_embeddings, _sigmoid_pad_body, 200)
    idx_c = indices.reshape(NUM_WORKERS, NUM_CHUNKS, CHUNK)
    pos_c = positions.reshape(NUM_WORKERS, NUM_CHUNKS, CHUNK)
    fill = jnp.zeros((NUM_WORKERS, NUM_CHUNKS, 8 - CHUNK), jnp.int32)
    packed = jnp.concatenate([idx_c, fill, pos_c, fill], axis=-1)
    return _sc_gather_mul(emb_pad, gate_pad, packed.reshape(NUM_WORKERS * NUM_CHUNKS * 16))


# speedup vs baseline: 1.7486x; 1.3642x over previous
import functools

import jax
import jax.numpy as jnp
from jax import lax
from jax.experimental import pallas as pl
from jax.experimental.pallas import tpu as pltpu
from jax.experimental.pallas import tpu_sc as plsc

NUM_KMERS = 4096
DIM = 10000
DIM_PAD = 10112
MAX_POSITIONS = 1000
BATCH = 1024
SEQ = 20
N = BATCH * SEQ

NUM_CORES = 2
NUM_SUBCORES = 16
NUM_WORKERS = NUM_CORES * NUM_SUBCORES
ROWS_PER_WORKER = N // NUM_WORKERS
CHUNK = 2
NUM_CHUNKS = ROWS_PER_WORKER // CHUNK
LANES = 16
VECS_PER_ROW = DIM // LANES


def _pad_body(in_ref, out_ref):
    out_ref[:, :DIM] = in_ref[...]
    out_ref[:, DIM:] = jnp.zeros((in_ref.shape[0], DIM_PAD - DIM), jnp.float32)


def _sigmoid_pad_body(in_ref, out_ref):
    out_ref[:, :DIM] = jax.nn.sigmoid(in_ref[...])
    out_ref[:, DIM:] = jnp.zeros((in_ref.shape[0], DIM_PAD - DIM), jnp.float32)


def _pad_table(table, body, rows_per_block):
    rows = table.shape[0]
    return pl.pallas_call(
        body,
        grid=(rows // rows_per_block,),
        in_specs=[pl.BlockSpec((rows_per_block, DIM), lambda i: (i, 0))],
        out_specs=pl.BlockSpec((rows_per_block, DIM_PAD), lambda i: (i, 0)),
        out_shape=jax.ShapeDtypeStruct((rows, DIM_PAD), jnp.float32),
    )(table)


_sc_mesh = plsc.VectorSubcoreMesh(core_axis_name="c", subcore_axis_name="s")


@functools.partial(
    pl.kernel,
    mesh=_sc_mesh,
    out_type=jax.ShapeDtypeStruct((BATCH, SEQ, DIM), jnp.float32),
    scratch_types=[
        pltpu.VMEM((NUM_CHUNKS * 16,), jnp.int32),
        pltpu.VMEM((CHUNK, DIM_PAD), jnp.float32),
        pltpu.VMEM((CHUNK, DIM_PAD), jnp.float32),
        pltpu.VMEM((CHUNK, DIM_PAD), jnp.float32),
        pltpu.VMEM((CHUNK, DIM_PAD), jnp.float32),
        pltpu.VMEM((CHUNK, DIM), jnp.float32),
        pltpu.SemaphoreType.DMA,
        pltpu.SemaphoreType.DMA,
        pltpu.SemaphoreType.DMA,
    ],
)
def _sc_gather_mul(emb_hbm, gate_hbm, packed_hbm, out_hbm,
                   packed_v, emb_v0, emb_v1, gate_v0, gate_v1, prod_v,
                   sem_g0, sem_g1, sem_s):
    wid = lax.axis_index("s") * NUM_CORES + lax.axis_index("c")
    base = wid * ROWS_PER_WORKER
    pltpu.sync_copy(packed_hbm.at[pl.ds(wid * NUM_CHUNKS * 16, NUM_CHUNKS * 16)],
                    packed_v)

    emb_bufs = (emb_v0, emb_v1)
    gate_bufs = (gate_v0, gate_v1)
    gather_sems = (sem_g0, sem_g1)

    def issue_gathers(g, b):
        pltpu.async_copy(emb_hbm.at[packed_v.at[pl.ds(g * 16, CHUNK)]],
                         emb_bufs[b], gather_sems[b])
        pltpu.async_copy(gate_hbm.at[packed_v.at[pl.ds(g * 16 + 8, CHUNK)]],
                         gate_bufs[b], gather_sems[b])

    def wait_gathers(g, b):
        pltpu.make_async_copy(emb_hbm.at[packed_v.at[pl.ds(g * 16, CHUNK)]],
                              emb_bufs[b], gather_sems[b]).wait()
        pltpu.make_async_copy(gate_hbm.at[packed_v.at[pl.ds(g * 16 + 8, CHUNK)]],
                              gate_bufs[b], gather_sems[b]).wait()

    def out_window(g):
        row = base + g * CHUNK
        return out_hbm.at[row // SEQ, pl.ds(row % SEQ, CHUNK)]

    issue_gathers(0, 0)

    def pair_body(i, carry):
        for b in range(2):
            g = i * 2 + b
            nxt = g + 1

            @pl.when(nxt < NUM_CHUNKS)
            def _():
                issue_gathers(nxt, 1 - b)

            @pl.when(g > 0)
            def _():
                pltpu.make_async_copy(prod_v, out_window(g - 1), sem_s).wait()

            wait_gathers(g, b)

            def vec_body(j, c):
                off = j * LANES
                for r in range(CHUNK):
                    prod_v[r, pl.ds(off, LANES)] = (
                        emb_bufs[b][r, pl.ds(off, LANES)]
                        * gate_bufs[b][r, pl.ds(off, LANES)]
                    )
                return c

            lax.fori_loop(0, VECS_PER_ROW, vec_body, 0)
            pltpu.async_copy(prod_v, out_window(g), sem_s)
        return carry

    lax.fori_loop(0, NUM_CHUNKS // 2, pair_body, 0)
    pltpu.make_async_copy(prod_v, out_window(NUM_CHUNKS - 1), sem_s).wait()


def kernel(embeddings, position_embeddings, indices, positions):
    emb_pad = _pad_table(embeddings, _pad_body, 128)
    gate_pad = _pad_table(position_embeddings, _sigmoid_pad_body, 200)
    idx_c = indices.reshape(NUM_WORKERS, NUM_CHUNKS, CHUNK)
    pos_c = positions.reshape(NUM_WORKERS, NUM_CHUNKS, CHUNK)
    fill = jnp.zeros((NUM_WORKERS, NUM_CHUNKS, 8 - CHUNK), jnp.int32)
    packed = jnp.concatenate([idx_c, fill, pos_c, fill], axis=-1)
    return _sc_gather_mul(emb_pad, gate_pad, packed.reshape(NUM_WORKERS * NUM_CHUNKS * 16))
